# Initial kernel scaffold; baseline (speedup 1.0000x reference)
import jax
import jax.numpy as jnp
from jax.experimental import pallas as pl


def kernel(adj, x, W, gc1_w, gc1_b, gc2_w, gc2_b, w_ih, w_hh, b_ih, b_hh, lin_w, lin_b):
    raise NotImplementedError("write your pallas kernel here")


# speedup vs baseline: 1.0933x; 1.0933x over previous
import jax
import jax.numpy as jnp
from jax.experimental import pallas as pl
from jax.experimental.pallas import tpu as pltpu

_N = 4096
_NFEAT = 128
_D1 = 32
_D2 = 32
_NMETA = 3
_HOUSE = 512
_SEQ = _N // _HOUSE
_BM = 512
_NBLK = _N // _BM


def _fused_kernel(w_ref, adj_ref, x_ref, gc1_w_ref, gc1_b_ref, gc2_w_ref,
                  gc2_b_ref, w_ih_t_ref, w_hh_t_ref, b_ih_ref, b_hh_ref,
                  lin_w_t_ref, lin_b_ref, out_ref, u_sc, v_sc, z_sc):
    i = pl.program_id(0)
    p = pl.program_id(1)
    r = pl.program_id(2)

    @pl.when((i == 0) & (p == 0) & (r == 0))
    def _init():
        u_sc[...] = jnp.dot(x_ref[...], gc1_w_ref[...],
                            preferred_element_type=jnp.float32)

    adj_blk = adj_ref[0]
    rows = pl.ds(r * _BM, _BM)

    @pl.when(p == 0)
    def _layer1():
        y1 = jnp.dot(adj_blk, u_sc[...], preferred_element_type=jnp.float32)
        h1 = jnp.maximum(y1 + gc1_b_ref[...], 0.0)
        v_sc[rows, :] = jnp.dot(h1, gc2_w_ref[...],
                                preferred_element_type=jnp.float32)

    @pl.when(p == 1)
    def _layer2():
        y2 = jnp.dot(adj_blk, v_sc[...], preferred_element_type=jnp.float32)
        h2 = jnp.maximum(y2 + gc2_b_ref[...], 0.0)
        contrib = w_ref[i, 0] * h2

        @pl.when(i == 0)
        def _():
            z_sc[rows, :] = contrib

        @pl.when(i > 0)
        def _():
            z_sc[rows, :] = z_sc[rows, :] + contrib

    @pl.when((i == _NMETA - 1) & (p == 1) & (r == _NBLK - 1))
    def _lstm_and_linear():
        z_sc[...] = jnp.maximum(z_sc[...], 0.0)
        w_ih_t = w_ih_t_ref[...]
        w_hh_t = w_hh_t_ref[...]
        b = b_ih_ref[...] + b_hh_ref[...]
        lin_w_t = lin_w_t_ref[...]
        lin_b = lin_b_ref[...]

        def step(t, carry):
            h, c = carry
            seq_rows = pl.ds(t * _HOUSE, _HOUSE)
            x_t = z_sc[seq_rows, :]
            gates = (jnp.dot(x_t, w_ih_t, preferred_element_type=jnp.float32)
                     + jnp.dot(h, w_hh_t, preferred_element_type=jnp.float32)
                     + b)
            ig = jax.nn.sigmoid(gates[:, 0 * _D2:1 * _D2])
            fg = jax.nn.sigmoid(gates[:, 1 * _D2:2 * _D2])
            gg = jnp.tanh(gates[:, 2 * _D2:3 * _D2])
            og = jax.nn.sigmoid(gates[:, 3 * _D2:4 * _D2])
            c_new = fg * c + ig * gg
            h_new = og * jnp.tanh(c_new)
            out_ref[seq_rows, :] = (
                jnp.dot(h_new, lin_w_t, preferred_element_type=jnp.float32)
                + lin_b)
            return h_new, c_new

        h0 = jnp.zeros((_HOUSE, _D2), dtype=jnp.float32)
        c0 = jnp.zeros((_HOUSE, _D2), dtype=jnp.float32)
        jax.lax.fori_loop(0, _SEQ, step, (h0, c0))


def kernel(adj, x, W, gc1_w, gc1_b, gc2_w, gc2_b, w_ih, w_hh, b_ih, b_hh,
           lin_w, lin_b):
    grid = (_NMETA, 2, _NBLK)
    out = pl.pallas_call(
        _fused_kernel,
        grid=grid,
        in_specs=[
            pl.BlockSpec(memory_space=pltpu.SMEM),
            pl.BlockSpec((1, _BM, _N), lambda i, p, r: (i, r, 0)),
            pl.BlockSpec((_N, _NFEAT), lambda i, p, r: (0, 0)),
            pl.BlockSpec((_NFEAT, _D1), lambda i, p, r: (0, 0)),
            pl.BlockSpec((1, _D1), lambda i, p, r: (0, 0)),
            pl.BlockSpec((_D1, _D2), lambda i, p, r: (0, 0)),
            pl.BlockSpec((1, _D2), lambda i, p, r: (0, 0)),
            pl.BlockSpec((_D2, 4 * _D2), lambda i, p, r: (0, 0)),
            pl.BlockSpec((_D2, 4 * _D2), lambda i, p, r: (0, 0)),
            pl.BlockSpec((1, 4 * _D2), lambda i, p, r: (0, 0)),
            pl.BlockSpec((1, 4 * _D2), lambda i, p, r: (0, 0)),
            pl.BlockSpec((_D2, 1), lambda i, p, r: (0, 0)),
            pl.BlockSpec((1, 1), lambda i, p, r: (0, 0)),
        ],
        out_specs=pl.BlockSpec((_N, 1), lambda i, p, r: (0, 0)),
        out_shape=jax.ShapeDtypeStruct((_N, 1), jnp.float32),
        scratch_shapes=[
            pltpu.VMEM((_N, _D1), jnp.float32),
            pltpu.VMEM((_N, _D2), jnp.float32),
            pltpu.VMEM((_N, _D2), jnp.float32),
        ],
        compiler_params=pltpu.CompilerParams(
            dimension_semantics=("arbitrary", "arbitrary", "arbitrary"),
        ),
    )(W, adj, x, gc1_w, gc1_b.reshape(1, _D1), gc2_w,
      gc2_b.reshape(1, _D2), w_ih.T, w_hh.T, b_ih.reshape(1, 4 * _D2),
      b_hh.reshape(1, 4 * _D2), lin_w.T, lin_b.reshape(1, 1))
    return out


# speedup vs baseline: 1.5111x; 1.3822x over previous
import jax
import jax.numpy as jnp
from jax.experimental import pallas as pl
from jax.experimental.pallas import tpu as pltpu

_N = 4096
_NFEAT = 128
_D1 = 32
_D2 = 32
_NMETA = 3
_HOUSE = 512
_SEQ = _N // _HOUSE
_RS = 2048
_CC = 1024
_NSTEP = 8
_U = 0
_V = 32
_Y = 64
_Z = 96


def _dot(a, b):
    return jnp.dot(a, b, preferred_element_type=jnp.float32)


def _fused_kernel(w_ref, adj_ref, x_ref, gc1_w_ref, gc1_b_ref, gc2_w_ref,
                  gc2_b_ref, w_ih_t_ref, w_hh_t_ref, b_ih_ref, b_hh_ref,
                  lin_w_t_ref, lin_b_ref, out_ref,
                  buf_sc, y1_sc, stash_a, stash_b, stash_c):
    i = pl.program_id(0)
    s = pl.program_id(1)

    @pl.when((i == 0) & (s == 0))
    def _init():
        buf_sc[:, _U:_U + _D1] = _dot(x_ref[...], gc1_w_ref[...])

    ablk = adj_ref[0]
    c = jnp.where(s < 4, (s + 2) % 4, s - 4)
    uc = buf_sc[pl.ds(c * _CC, _CC), _U:_U + _D1]

    part = _dot(ablk, uc)

    @pl.when((s == 0) | (s == 4))
    def _y1_init():
        y1_sc[...] = part

    @pl.when((s != 0) & (s != 4))
    def _y1_acc():
        y1_sc[...] = y1_sc[...] + part

    @pl.when(s == 0)
    def _():
        stash_a[...] = ablk

    @pl.when(s == 1)
    def _():
        stash_b[...] = ablk

    @pl.when((s == 2) | (s == 6))
    def _():
        stash_c[...] = ablk

    @pl.when(s == 3)
    def _strip0_done():
        v0 = _dot(jnp.maximum(y1_sc[...] + gc1_b_ref[...], 0.0),
                  gc2_w_ref[...])
        buf_sc[0:_RS, _V:_V + _D2] = v0
        buf_sc[0:_RS, _Y:_Y + _D2] = (_dot(stash_c[...], v0[0:_CC, :])
                                      + _dot(ablk, v0[_CC:_RS, :]))

    @pl.when(s == 4)
    def _():
        buf_sc[_RS:_N, _Y:_Y + _D2] = _dot(
            ablk, buf_sc[0:_CC, _V:_V + _D2])

    @pl.when(s == 5)
    def _():
        buf_sc[_RS:_N, _Y:_Y + _D2] = buf_sc[_RS:_N, _Y:_Y + _D2] + _dot(
            ablk, buf_sc[_CC:_RS, _V:_V + _D2])

    @pl.when(s == 7)
    def _strip1_done():
        v1 = _dot(jnp.maximum(y1_sc[...] + gc1_b_ref[...], 0.0),
                  gc2_w_ref[...])
        buf_sc[_RS:_N, _V:_V + _D2] = v1
        buf_sc[_RS:_N, _Y:_Y + _D2] = (buf_sc[_RS:_N, _Y:_Y + _D2]
                                       + _dot(stash_c[...], v1[0:_CC, :])
                                       + _dot(ablk, v1[_CC:_RS, :]))
        buf_sc[0:_RS, _Y:_Y + _D2] = (buf_sc[0:_RS, _Y:_Y + _D2]
                                      + _dot(stash_a[...], v1[0:_CC, :])
                                      + _dot(stash_b[...], v1[_CC:_RS, :]))
        h2 = jnp.maximum(buf_sc[:, _Y:_Y + _D2] + gc2_b_ref[...], 0.0)
        contrib = w_ref[i, 0] * h2

        @pl.when(i == 0)
        def _():
            buf_sc[:, _Z:_Z + _D2] = contrib

        @pl.when(i > 0)
        def _():
            buf_sc[:, _Z:_Z + _D2] = buf_sc[:, _Z:_Z + _D2] + contrib

    @pl.when((i == _NMETA - 1) & (s == _NSTEP - 1))
    def _lstm_and_linear():
        buf_sc[:, _Z:_Z + _D2] = jnp.maximum(buf_sc[:, _Z:_Z + _D2], 0.0)
        w_ih_t = w_ih_t_ref[...]
        w_hh_t = w_hh_t_ref[...]
        b = b_ih_ref[...] + b_hh_ref[...]
        lin_w_t = lin_w_t_ref[...]
        lin_b = lin_b_ref[...]

        def step(t, carry):
            h, cc = carry
            seq_rows = pl.ds(t * _HOUSE, _HOUSE)
            x_t = buf_sc[seq_rows, _Z:_Z + _D2]
            gates = _dot(x_t, w_ih_t) + _dot(h, w_hh_t) + b
            ig = jax.nn.sigmoid(gates[:, 0 * _D2:1 * _D2])
            fg = jax.nn.sigmoid(gates[:, 1 * _D2:2 * _D2])
            gg = jnp.tanh(gates[:, 2 * _D2:3 * _D2])
            og = jax.nn.sigmoid(gates[:, 3 * _D2:4 * _D2])
            c_new = fg * cc + ig * gg
            h_new = og * jnp.tanh(c_new)
            out_ref[seq_rows, :] = _dot(h_new, lin_w_t) + lin_b
            return h_new, c_new

        h0 = jnp.zeros((_HOUSE, _D2), dtype=jnp.float32)
        c0 = jnp.zeros((_HOUSE, _D2), dtype=jnp.float32)
        jax.lax.fori_loop(0, _SEQ, step, (h0, c0))


def _adj_index(i, s):
    return (i, s // 4, jnp.where(s < 4, (s + 2) % 4, s - 4))


def kernel(adj, x, W, gc1_w, gc1_b, gc2_w, gc2_b, w_ih, w_hh, b_ih, b_hh,
           lin_w, lin_b):
    grid = (_NMETA, _NSTEP)
    out = pl.pallas_call(
        _fused_kernel,
        grid=grid,
        in_specs=[
            pl.BlockSpec(memory_space=pltpu.SMEM),
            pl.BlockSpec((1, _RS, _CC), _adj_index),
            pl.BlockSpec((_N, _NFEAT), lambda i, s: (0, 0)),
            pl.BlockSpec((_NFEAT, _D1), lambda i, s: (0, 0)),
            pl.BlockSpec((1, _D1), lambda i, s: (0, 0)),
            pl.BlockSpec((_D1, _D2), lambda i, s: (0, 0)),
            pl.BlockSpec((1, _D2), lambda i, s: (0, 0)),
            pl.BlockSpec((_D2, 4 * _D2), lambda i, s: (0, 0)),
            pl.BlockSpec((_D2, 4 * _D2), lambda i, s: (0, 0)),
            pl.BlockSpec((1, 4 * _D2), lambda i, s: (0, 0)),
            pl.BlockSpec((1, 4 * _D2), lambda i, s: (0, 0)),
            pl.BlockSpec((_D2, 1), lambda i, s: (0, 0)),
            pl.BlockSpec((1, 1), lambda i, s: (0, 0)),
        ],
        out_specs=pl.BlockSpec((_N, 1), lambda i, s: (0, 0)),
        out_shape=jax.ShapeDtypeStruct((_N, 1), jnp.float32),
        scratch_shapes=[
            pltpu.VMEM((_N, 128), jnp.float32),
            pltpu.VMEM((_RS, _D1), jnp.float32),
            pltpu.VMEM((_RS, _CC), jnp.float32),
            pltpu.VMEM((_RS, _CC), jnp.float32),
            pltpu.VMEM((_RS, _CC), jnp.float32),
        ],
        compiler_params=pltpu.CompilerParams(
            dimension_semantics=("arbitrary", "arbitrary"),
            vmem_limit_bytes=100 * 1024 * 1024,
        ),
    )(W, adj, x, gc1_w, gc1_b.reshape(1, _D1), gc2_w,
      gc2_b.reshape(1, _D2), w_ih.T, w_hh.T, b_ih.reshape(1, 4 * _D2),
      b_hh.reshape(1, 4 * _D2), lin_w.T, lin_b.reshape(1, 1))
    return out


# speedup vs baseline: 1.5381x; 1.0178x over previous
import jax
import jax.numpy as jnp
from jax.experimental import pallas as pl
from jax.experimental.pallas import tpu as pltpu

_N = 4096
_NFEAT = 128
_D1 = 32
_D2 = 32
_NMETA = 3
_HOUSE = 512
_SEQ = _N // _HOUSE
_RS = 2048
_CC = 1024
_NSTEP = 8
_U = 0
_V = 32
_Y = 64
_Z = 96


def _dot(a, b):
    return jnp.dot(a, b, preferred_element_type=jnp.float32)


def _bdot(a_bf16, b_f32):
    return jnp.dot(a_bf16, b_f32.astype(jnp.bfloat16),
                   preferred_element_type=jnp.float32)


def _fused_kernel(w_ref, adj_ref, x_ref, gc1_w_ref, gc1_b_ref, gc2_w_ref,
                  gc2_b_ref, w_ih_t_ref, w_hh_t_ref, b_ih_ref, b_hh_ref,
                  lin_w_t_ref, lin_b_ref, out_ref,
                  buf_sc, y1_sc, stash_a, stash_b, stash_c):
    i = pl.program_id(0)
    s = pl.program_id(1)

    @pl.when((i == 0) & (s == 0))
    def _init():
        buf_sc[:, _U:_U + _D1] = _dot(x_ref[...], gc1_w_ref[...])

    abf = adj_ref[0].astype(jnp.bfloat16)
    c = jnp.where(s < 4, (s + 2) % 4, s - 4)
    uc = buf_sc[pl.ds(c * _CC, _CC), _U:_U + _D1]

    part = _bdot(abf, uc)

    @pl.when((s == 0) | (s == 4))
    def _y1_init():
        y1_sc[...] = part

    @pl.when((s != 0) & (s != 4))
    def _y1_acc():
        y1_sc[...] = y1_sc[...] + part

    @pl.when(s == 0)
    def _():
        stash_a[...] = abf

    @pl.when(s == 1)
    def _():
        stash_b[...] = abf

    @pl.when((s == 2) | (s == 6))
    def _():
        stash_c[...] = abf

    @pl.when(s == 3)
    def _strip0_done():
        v0 = _dot(jnp.maximum(y1_sc[...] + gc1_b_ref[...], 0.0),
                  gc2_w_ref[...])
        buf_sc[0:_RS, _V:_V + _D2] = v0
        buf_sc[0:_RS, _Y:_Y + _D2] = (_bdot(stash_c[...], v0[0:_CC, :])
                                      + _bdot(abf, v0[_CC:_RS, :]))

    @pl.when(s == 4)
    def _():
        buf_sc[_RS:_N, _Y:_Y + _D2] = _bdot(
            abf, buf_sc[0:_CC, _V:_V + _D2])

    @pl.when(s == 5)
    def _():
        buf_sc[_RS:_N, _Y:_Y + _D2] = buf_sc[_RS:_N, _Y:_Y + _D2] + _bdot(
            abf, buf_sc[_CC:_RS, _V:_V + _D2])

    @pl.when(s == 7)
    def _strip1_done():
        v1 = _dot(jnp.maximum(y1_sc[...] + gc1_b_ref[...], 0.0),
                  gc2_w_ref[...])
        buf_sc[_RS:_N, _V:_V + _D2] = v1
        buf_sc[_RS:_N, _Y:_Y + _D2] = (buf_sc[_RS:_N, _Y:_Y + _D2]
                                       + _bdot(stash_c[...], v1[0:_CC, :])
                                       + _bdot(abf, v1[_CC:_RS, :]))
        buf_sc[0:_RS, _Y:_Y + _D2] = (buf_sc[0:_RS, _Y:_Y + _D2]
                                      + _bdot(stash_a[...], v1[0:_CC, :])
                                      + _bdot(stash_b[...], v1[_CC:_RS, :]))
        h2 = jnp.maximum(buf_sc[:, _Y:_Y + _D2] + gc2_b_ref[...], 0.0)
        contrib = w_ref[i, 0] * h2

        @pl.when(i == 0)
        def _():
            buf_sc[:, _Z:_Z + _D2] = contrib

        @pl.when(i > 0)
        def _():
            buf_sc[:, _Z:_Z + _D2] = buf_sc[:, _Z:_Z + _D2] + contrib

    @pl.when((i == _NMETA - 1) & (s == _NSTEP - 1))
    def _lstm_and_linear():
        buf_sc[:, _Z:_Z + _D2] = jnp.maximum(buf_sc[:, _Z:_Z + _D2], 0.0)
        w_ih_t = w_ih_t_ref[...]
        w_hh_t = w_hh_t_ref[...]
        b = b_ih_ref[...] + b_hh_ref[...]
        lin_w_t = lin_w_t_ref[...]
        lin_b = lin_b_ref[...]

        def step(t, carry):
            h, cc = carry
            seq_rows = pl.ds(t * _HOUSE, _HOUSE)
            x_t = buf_sc[seq_rows, _Z:_Z + _D2]
            gates = _dot(x_t, w_ih_t) + _dot(h, w_hh_t) + b
            ig = jax.nn.sigmoid(gates[:, 0 * _D2:1 * _D2])
            fg = jax.nn.sigmoid(gates[:, 1 * _D2:2 * _D2])
            gg = jnp.tanh(gates[:, 2 * _D2:3 * _D2])
            og = jax.nn.sigmoid(gates[:, 3 * _D2:4 * _D2])
            c_new = fg * cc + ig * gg
            h_new = og * jnp.tanh(c_new)
            out_ref[seq_rows, :] = _dot(h_new, lin_w_t) + lin_b
            return h_new, c_new

        h0 = jnp.zeros((_HOUSE, _D2), dtype=jnp.float32)
        c0 = jnp.zeros((_HOUSE, _D2), dtype=jnp.float32)
        jax.lax.fori_loop(0, _SEQ, step, (h0, c0))


def _adj_index(i, s):
    return (i, s // 4, jnp.where(s < 4, (s + 2) % 4, s - 4))


def kernel(adj, x, W, gc1_w, gc1_b, gc2_w, gc2_b, w_ih, w_hh, b_ih, b_hh,
           lin_w, lin_b):
    grid = (_NMETA, _NSTEP)
    out = pl.pallas_call(
        _fused_kernel,
        grid=grid,
        in_specs=[
            pl.BlockSpec(memory_space=pltpu.SMEM),
            pl.BlockSpec((1, _RS, _CC), _adj_index),
            pl.BlockSpec((_N, _NFEAT), lambda i, s: (0, 0)),
            pl.BlockSpec((_NFEAT, _D1), lambda i, s: (0, 0)),
            pl.BlockSpec((1, _D1), lambda i, s: (0, 0)),
            pl.BlockSpec((_D1, _D2), lambda i, s: (0, 0)),
            pl.BlockSpec((1, _D2), lambda i, s: (0, 0)),
            pl.BlockSpec((_D2, 4 * _D2), lambda i, s: (0, 0)),
            pl.BlockSpec((_D2, 4 * _D2), lambda i, s: (0, 0)),
            pl.BlockSpec((1, 4 * _D2), lambda i, s: (0, 0)),
            pl.BlockSpec((1, 4 * _D2), lambda i, s: (0, 0)),
            pl.BlockSpec((_D2, 1), lambda i, s: (0, 0)),
            pl.BlockSpec((1, 1), lambda i, s: (0, 0)),
        ],
        out_specs=pl.BlockSpec((_N, 1), lambda i, s: (0, 0)),
        out_shape=jax.ShapeDtypeStruct((_N, 1), jnp.float32),
        scratch_shapes=[
            pltpu.VMEM((_N, 128), jnp.float32),
            pltpu.VMEM((_RS, _D1), jnp.float32),
            pltpu.VMEM((_RS, _CC), jnp.bfloat16),
            pltpu.VMEM((_RS, _CC), jnp.bfloat16),
            pltpu.VMEM((_RS, _CC), jnp.bfloat16),
        ],
        compiler_params=pltpu.CompilerParams(
            dimension_semantics=("arbitrary", "arbitrary"),
            vmem_limit_bytes=100 * 1024 * 1024,
        ),
    )(W, adj, x, gc1_w, gc1_b.reshape(1, _D1), gc2_w,
      gc2_b.reshape(1, _D2), w_ih.T, w_hh.T, b_ih.reshape(1, 4 * _D2),
      b_hh.reshape(1, 4 * _D2), lin_w.T, lin_b.reshape(1, 1))
    return out


# speedup vs baseline: 1.5390x; 1.0006x over previous
import jax
import jax.numpy as jnp
from jax.experimental import pallas as pl
from jax.experimental.pallas import tpu as pltpu

_N = 4096
_NFEAT = 128
_D1 = 32
_D2 = 32
_NMETA = 3
_HOUSE = 512
_SEQ = _N // _HOUSE
_RS = 512
_NSTRIP = _N // _RS
_U = 0
_VP = 32
_VC = 64
_Z = 96


def _dot(a, b):
    return jnp.dot(a, b, preferred_element_type=jnp.float32)


def _b16(t):
    return t.astype(jnp.bfloat16)


def _fused_kernel(w_ref, adj_ref, x_ref, gc1_w_ref, gc1_b_ref, gc2_w_ref,
                  gc2_b_ref, w_ih_t_ref, w_hh_t_ref, b_ih_ref, b_hh_ref,
                  lin_w_t_ref, lin_b_ref, out_ref, buf_sc, stash_sc):
    i = pl.program_id(0)
    r = pl.program_id(1)
    rows = pl.ds(r * _RS, _RS)

    @pl.when((i == 0) & (r == 0))
    def _init():
        buf_sc[:, _U:_U + _D1] = _dot(x_ref[...], gc1_w_ref[...])

    @pl.when((i > 0) & (r == 0))
    def _promote_v():
        buf_sc[:, _VP:_VP + _D2] = buf_sc[:, _VC:_VC + _D2]

    @pl.when(i > 0)
    def _layer2_prev_meta():
        y2 = _dot(stash_sc[rows, :], _b16(buf_sc[:, _VP:_VP + _D2]))
        h2 = jnp.maximum(y2 + gc2_b_ref[...], 0.0)
        contrib = w_ref[i - 1, 0] * h2

        @pl.when(i == 1)
        def _():
            buf_sc[rows, _Z:_Z + _D2] = contrib

        @pl.when(i > 1)
        def _():
            buf_sc[rows, _Z:_Z + _D2] = buf_sc[rows, _Z:_Z + _D2] + contrib

    @pl.when(i < _NMETA)
    def _layer1_cur_meta():
        abf = _b16(adj_ref[0])
        y1 = _dot(abf, _b16(buf_sc[:, _U:_U + _D1]))
        v = _dot(jnp.maximum(y1 + gc1_b_ref[...], 0.0), gc2_w_ref[...])
        buf_sc[rows, _VC:_VC + _D2] = v
        stash_sc[rows, :] = abf

    @pl.when((i == _NMETA) & (r == _NSTRIP - 1))
    def _lstm_and_linear():
        buf_sc[:, _Z:_Z + _D2] = jnp.maximum(buf_sc[:, _Z:_Z + _D2], 0.0)
        w_ih_t = w_ih_t_ref[...]
        w_hh_t = w_hh_t_ref[...]
        b = b_ih_ref[...] + b_hh_ref[...]
        lin_w_t = lin_w_t_ref[...]
        lin_b = lin_b_ref[...]

        def step(t, carry):
            h, cc = carry
            seq_rows = pl.ds(t * _HOUSE, _HOUSE)
            x_t = buf_sc[seq_rows, _Z:_Z + _D2]
            gates = _dot(x_t, w_ih_t) + _dot(h, w_hh_t) + b
            ig = jax.nn.sigmoid(gates[:, 0 * _D2:1 * _D2])
            fg = jax.nn.sigmoid(gates[:, 1 * _D2:2 * _D2])
            gg = jnp.tanh(gates[:, 2 * _D2:3 * _D2])
            og = jax.nn.sigmoid(gates[:, 3 * _D2:4 * _D2])
            c_new = fg * cc + ig * gg
            h_new = og * jnp.tanh(c_new)
            out_ref[seq_rows, :] = _dot(h_new, lin_w_t) + lin_b
            return h_new, c_new

        h0 = jnp.zeros((_HOUSE, _D2), dtype=jnp.float32)
        c0 = jnp.zeros((_HOUSE, _D2), dtype=jnp.float32)
        jax.lax.fori_loop(0, _SEQ, step, (h0, c0))


def _adj_index(i, r):
    return (jnp.minimum(i, _NMETA - 1),
            jnp.where(i < _NMETA, r, _NSTRIP - 1), 0)


def kernel(adj, x, W, gc1_w, gc1_b, gc2_w, gc2_b, w_ih, w_hh, b_ih, b_hh,
           lin_w, lin_b):
    grid = (_NMETA + 1, _NSTRIP)
    out = pl.pallas_call(
        _fused_kernel,
        grid=grid,
        in_specs=[
            pl.BlockSpec(memory_space=pltpu.SMEM),
            pl.BlockSpec((1, _RS, _N), _adj_index),
            pl.BlockSpec((_N, _NFEAT), lambda i, r: (0, 0)),
            pl.BlockSpec((_NFEAT, _D1), lambda i, r: (0, 0)),
            pl.BlockSpec((1, _D1), lambda i, r: (0, 0)),
            pl.BlockSpec((_D1, _D2), lambda i, r: (0, 0)),
            pl.BlockSpec((1, _D2), lambda i, r: (0, 0)),
            pl.BlockSpec((_D2, 4 * _D2), lambda i, r: (0, 0)),
            pl.BlockSpec((_D2, 4 * _D2), lambda i, r: (0, 0)),
            pl.BlockSpec((1, 4 * _D2), lambda i, r: (0, 0)),
            pl.BlockSpec((1, 4 * _D2), lambda i, r: (0, 0)),
            pl.BlockSpec((_D2, 1), lambda i, r: (0, 0)),
            pl.BlockSpec((1, 1), lambda i, r: (0, 0)),
        ],
        out_specs=pl.BlockSpec((_N, 1), lambda i, r: (0, 0)),
        out_shape=jax.ShapeDtypeStruct((_N, 1), jnp.float32),
        scratch_shapes=[
            pltpu.VMEM((_N, 128), jnp.float32),
            pltpu.VMEM((_N, _N), jnp.bfloat16),
        ],
        compiler_params=pltpu.CompilerParams(
            dimension_semantics=("arbitrary", "arbitrary"),
            vmem_limit_bytes=100 * 1024 * 1024,
        ),
    )(W, adj, x, gc1_w, gc1_b.reshape(1, _D1), gc2_w,
      gc2_b.reshape(1, _D2), w_ih.T, w_hh.T, b_ih.reshape(1, 4 * _D2),
      b_hh.reshape(1, 4 * _D2), lin_w.T, lin_b.reshape(1, 1))
    return out


# speedup vs baseline: 1.6267x; 1.0570x over previous
import jax
import jax.numpy as jnp
from jax.experimental import pallas as pl
from jax.experimental.pallas import tpu as pltpu

_N = 4096
_NFEAT = 128
_D1 = 32
_D2 = 32
_NMETA = 3
_HOUSE = 512
_SEQ = _N // _HOUSE
_RS = 512
_NSTRIP = _N // _RS


def _dot(a, b):
    return jnp.dot(a, b, preferred_element_type=jnp.float32)


def _b16(t):
    return t.astype(jnp.bfloat16)


def _fused_kernel(w_ref, adj_ref, x_ref, gc1_w_ref, gc1_b_ref, gc2_w_ref,
                  gc2_b_ref, w_ih_t_ref, w_hh_t_ref, b_ih_ref, b_hh_ref,
                  lin_w_t_ref, lin_b_ref, out_ref,
                  u_sc, vp_sc, vc_sc, z_sc, stash_sc):
    i = pl.program_id(0)
    r = pl.program_id(1)
    rows = pl.ds(r * _RS, _RS)

    @pl.when((i == 0) & (r == 0))
    def _init():
        u_sc[...] = _b16(_dot(x_ref[...], gc1_w_ref[...]))

    @pl.when((i > 0) & (r == 0))
    def _promote_v():
        vp_sc[...] = _b16(vc_sc[...])

    @pl.when(i > 0)
    def _layer2_prev_meta():
        y2 = _dot(stash_sc[rows, :], vp_sc[...])
        h2 = jnp.maximum(y2 + gc2_b_ref[...], 0.0)
        contrib = w_ref[i - 1, 0] * h2

        @pl.when(i == 1)
        def _():
            z_sc[rows, :] = contrib

        @pl.when(i > 1)
        def _():
            z_sc[rows, :] = z_sc[rows, :] + contrib

    @pl.when(i < _NMETA)
    def _layer1_cur_meta():
        abf = _b16(adj_ref[0])
        y1 = _dot(abf, u_sc[...])
        v = _dot(jnp.maximum(y1 + gc1_b_ref[...], 0.0), gc2_w_ref[...])
        vc_sc[rows, :] = v
        stash_sc[rows, :] = abf

    @pl.when((i == _NMETA) & (r == _NSTRIP - 1))
    def _lstm_and_linear():
        z_sc[...] = jnp.maximum(z_sc[...], 0.0)
        w_ih_t = w_ih_t_ref[...]
        w_hh_t = w_hh_t_ref[...]
        b = b_ih_ref[...] + b_hh_ref[...]
        lin_w_t = lin_w_t_ref[...]
        lin_b = lin_b_ref[...]

        def step(t, carry):
            h, cc = carry
            seq_rows = pl.ds(t * _HOUSE, _HOUSE)
            x_t = z_sc[seq_rows, :]
            gates = _dot(x_t, w_ih_t) + _dot(h, w_hh_t) + b
            ig = jax.nn.sigmoid(gates[:, 0 * _D2:1 * _D2])
            fg = jax.nn.sigmoid(gates[:, 1 * _D2:2 * _D2])
            gg = jnp.tanh(gates[:, 2 * _D2:3 * _D2])
            og = jax.nn.sigmoid(gates[:, 3 * _D2:4 * _D2])
            c_new = fg * cc + ig * gg
            h_new = og * jnp.tanh(c_new)
            out_ref[seq_rows, :] = _dot(h_new, lin_w_t) + lin_b
            return h_new, c_new

        h0 = jnp.zeros((_HOUSE, _D2), dtype=jnp.float32)
        c0 = jnp.zeros((_HOUSE, _D2), dtype=jnp.float32)
        jax.lax.fori_loop(0, _SEQ, step, (h0, c0))


def _adj_index(i, r):
    return (jnp.minimum(i, _NMETA - 1),
            jnp.where(i < _NMETA, r, _NSTRIP - 1), 0)


def kernel(adj, x, W, gc1_w, gc1_b, gc2_w, gc2_b, w_ih, w_hh, b_ih, b_hh,
           lin_w, lin_b):
    grid = (_NMETA + 1, _NSTRIP)
    out = pl.pallas_call(
        _fused_kernel,
        grid=grid,
        in_specs=[
            pl.BlockSpec(memory_space=pltpu.SMEM),
            pl.BlockSpec((1, _RS, _N), _adj_index),
            pl.BlockSpec((_N, _NFEAT), lambda i, r: (0, 0)),
            pl.BlockSpec((_NFEAT, _D1), lambda i, r: (0, 0)),
            pl.BlockSpec((1, _D1), lambda i, r: (0, 0)),
            pl.BlockSpec((_D1, _D2), lambda i, r: (0, 0)),
            pl.BlockSpec((1, _D2), lambda i, r: (0, 0)),
            pl.BlockSpec((_D2, 4 * _D2), lambda i, r: (0, 0)),
            pl.BlockSpec((_D2, 4 * _D2), lambda i, r: (0, 0)),
            pl.BlockSpec((1, 4 * _D2), lambda i, r: (0, 0)),
            pl.BlockSpec((1, 4 * _D2), lambda i, r: (0, 0)),
            pl.BlockSpec((_D2, 1), lambda i, r: (0, 0)),
            pl.BlockSpec((1, 1), lambda i, r: (0, 0)),
        ],
        out_specs=pl.BlockSpec((_N, 1), lambda i, r: (0, 0)),
        out_shape=jax.ShapeDtypeStruct((_N, 1), jnp.float32),
        scratch_shapes=[
            pltpu.VMEM((_N, _D1), jnp.bfloat16),
            pltpu.VMEM((_N, _D2), jnp.bfloat16),
            pltpu.VMEM((_N, _D2), jnp.float32),
            pltpu.VMEM((_N, _D2), jnp.float32),
            pltpu.VMEM((_N, _N), jnp.bfloat16),
        ],
        compiler_params=pltpu.CompilerParams(
            dimension_semantics=("arbitrary", "arbitrary"),
            vmem_limit_bytes=100 * 1024 * 1024,
        ),
    )(W, adj, x, gc1_w, gc1_b.reshape(1, _D1), gc2_w,
      gc2_b.reshape(1, _D2), w_ih.T, w_hh.T, b_ih.reshape(1, 4 * _D2),
      b_hh.reshape(1, 4 * _D2), lin_w.T, lin_b.reshape(1, 1))
    return out


# speedup vs baseline: 1.6680x; 1.0254x over previous
import jax
import jax.numpy as jnp
from jax.experimental import pallas as pl
from jax.experimental.pallas import tpu as pltpu

_N = 4096
_NFEAT = 128
_D1 = 32
_D2 = 32
_NMETA = 3
_HOUSE = 512
_SEQ = _N // _HOUSE
_RS = 512
_NSTRIP = _N // _RS


def _dot(a, b):
    return jnp.dot(a, b, preferred_element_type=jnp.float32)


def _b16(t):
    return t.astype(jnp.bfloat16)


def _fused_kernel(w_ref, adj_ref, x_ref, gc1_w_ref, gc1_b_ref, gc2_w_ref,
                  gc2_b_ref, w_ih_t_ref, w_hh_t_ref, b_ih_ref, b_hh_ref,
                  lin_w_t_ref, lin_b_ref, out_ref,
                  u_sc, vp_sc, vc_sc, z_sc, stash_sc):
    i = pl.program_id(0)
    r = pl.program_id(1)
    rows = pl.ds(r * _RS, _RS)

    @pl.when((i == 0) & (r == 0))
    def _init():
        u_sc[...] = _b16(_dot(x_ref[...], gc1_w_ref[...]))

    @pl.when((i > 0) & (r == 0))
    def _promote_v():
        vp_sc[...] = _b16(vc_sc[...])

    @pl.when((i > 0) & (i < _NMETA))
    def _layer2_prev_meta():
        y2 = _dot(stash_sc[rows, :], vp_sc[...])
        h2 = jnp.maximum(y2 + gc2_b_ref[...], 0.0)
        contrib = w_ref[i - 1, 0] * h2

        @pl.when(i == 1)
        def _():
            z_sc[rows, :] = contrib

        @pl.when(i > 1)
        def _():
            z_sc[rows, :] = z_sc[rows, :] + contrib

    @pl.when((i == _NMETA) & (r < 2))
    def _layer2_drain():
        half = pl.ds(r * (_N // 2), _N // 2)
        y2 = _dot(stash_sc[half, :], vp_sc[...])
        h2 = jnp.maximum(y2 + gc2_b_ref[...], 0.0)
        z_sc[half, :] = z_sc[half, :] + w_ref[_NMETA - 1, 0] * h2

    @pl.when(i < _NMETA)
    def _layer1_cur_meta():
        abf = _b16(adj_ref[0])
        y1 = _dot(abf, u_sc[...])
        v = _dot(jnp.maximum(y1 + gc1_b_ref[...], 0.0), gc2_w_ref[...])
        vc_sc[rows, :] = v
        stash_sc[rows, :] = abf

    @pl.when((i == _NMETA) & (r == 2))
    def _lstm_and_linear():
        z_sc[...] = jnp.maximum(z_sc[...], 0.0)
        w_ih_t = w_ih_t_ref[...]
        w_hh_t = w_hh_t_ref[...]
        b = b_ih_ref[...] + b_hh_ref[...]
        lin_w_t = lin_w_t_ref[...]
        lin_b = lin_b_ref[...]

        def step(t, carry):
            h, cc = carry
            seq_rows = pl.ds(t * _HOUSE, _HOUSE)
            gates = _dot(z_sc[seq_rows, :], w_ih_t) + _dot(h, w_hh_t) + b
            ig = jax.nn.sigmoid(gates[:, 0 * _D2:1 * _D2])
            fg = jax.nn.sigmoid(gates[:, 1 * _D2:2 * _D2])
            gg = jnp.tanh(gates[:, 2 * _D2:3 * _D2])
            og = jax.nn.sigmoid(gates[:, 3 * _D2:4 * _D2])
            c_new = fg * cc + ig * gg
            h_new = og * jnp.tanh(c_new)
            out_ref[seq_rows, :] = _dot(h_new, lin_w_t) + lin_b
            return h_new, c_new

        h0 = jnp.zeros((_HOUSE, _D2), dtype=jnp.float32)
        c0 = jnp.zeros((_HOUSE, _D2), dtype=jnp.float32)
        jax.lax.fori_loop(0, _SEQ, step, (h0, c0))


def _adj_index(i, r):
    return (jnp.minimum(i, _NMETA - 1),
            jnp.where(i < _NMETA, r, _NSTRIP - 1), 0)


def kernel(adj, x, W, gc1_w, gc1_b, gc2_w, gc2_b, w_ih, w_hh, b_ih, b_hh,
           lin_w, lin_b):
    grid = (_NMETA + 1, _NSTRIP)
    out = pl.pallas_call(
        _fused_kernel,
        grid=grid,
        in_specs=[
            pl.BlockSpec(memory_space=pltpu.SMEM),
            pl.BlockSpec((1, _RS, _N), _adj_index),
            pl.BlockSpec((_N, _NFEAT), lambda i, r: (0, 0)),
            pl.BlockSpec((_NFEAT, _D1), lambda i, r: (0, 0)),
            pl.BlockSpec((1, _D1), lambda i, r: (0, 0)),
            pl.BlockSpec((_D1, _D2), lambda i, r: (0, 0)),
            pl.BlockSpec((1, _D2), lambda i, r: (0, 0)),
            pl.BlockSpec((_D2, 4 * _D2), lambda i, r: (0, 0)),
            pl.BlockSpec((_D2, 4 * _D2), lambda i, r: (0, 0)),
            pl.BlockSpec((1, 4 * _D2), lambda i, r: (0, 0)),
            pl.BlockSpec((1, 4 * _D2), lambda i, r: (0, 0)),
            pl.BlockSpec((_D2, 1), lambda i, r: (0, 0)),
            pl.BlockSpec((1, 1), lambda i, r: (0, 0)),
        ],
        out_specs=pl.BlockSpec((_N, 1), lambda i, r: (0, 0)),
        out_shape=jax.ShapeDtypeStruct((_N, 1), jnp.float32),
        scratch_shapes=[
            pltpu.VMEM((_N, _D1), jnp.bfloat16),
            pltpu.VMEM((_N, _D2), jnp.bfloat16),
            pltpu.VMEM((_N, _D2), jnp.float32),
            pltpu.VMEM((_N, _D2), jnp.float32),
            pltpu.VMEM((_N, _N), jnp.bfloat16),
        ],
        compiler_params=pltpu.CompilerParams(
            dimension_semantics=("arbitrary", "arbitrary"),
            vmem_limit_bytes=100 * 1024 * 1024,
        ),
    )(W, adj, x, gc1_w, gc1_b.reshape(1, _D1), gc2_w,
      gc2_b.reshape(1, _D2), w_ih.T, w_hh.T, b_ih.reshape(1, 4 * _D2),
      b_hh.reshape(1, 4 * _D2), lin_w.T, lin_b.reshape(1, 1))
    return out
